# baseline (speedup 1.0000x reference)
import numpy as np
import jax
import jax.numpy as jnp
from jax import lax
from jax.experimental import pallas as pl
from jax.experimental.pallas import tpu as pltpu

_V = 100000
_R = 8
_NROWS = 512
_CH = 1024
_NV = 98

_ROT = ((13, 15, 26, 6), (17, 29, 16, 24),
        (13, 15, 26, 6), (17, 29, 16, 24),
        (13, 15, 26, 6))

_KS = (np.uint32(0), np.uint32(42),
       np.uint32(np.uint32(42) ^ np.uint32(0x1BD11BDA)))

_TINY = np.float32(np.finfo(np.float32).tiny)
_NEGINF = np.float32(-np.inf)


def _rotl(x, r):
    return (x << np.uint32(r)) | (x >> np.uint32(32 - r))


def _gumbel_bits(flat_u32):
    t0 = flat_u32 + _KS[1]
    x0 = t0
    x1 = _rotl(t0, 13) ^ t0
    first = True
    for g in range(5):
        for r in _ROT[g][1 if first else 0:]:
            x0 = x0 + x1
            x1 = _rotl(x1, r) ^ x0
        first = False
        x0 = x0 + _KS[(g + 1) % 3]
        x1 = x1 + _KS[(g + 2) % 3] + np.uint32(g + 1)
    return x0 ^ x1


def _body(lg_ref, samp_ref, lp_ref, tv_ref, col_ref, lm_ref, s_ref, lgw_ref):
    i = pl.program_id(0)
    j = pl.program_id(1)

    @pl.when(j == 0)
    def _init():
        tv_ref[...] = jnp.full((_R, _CH), _NEGINF, jnp.float32)
        col_ref[...] = jnp.zeros((_R, _CH), jnp.int32)
        lm_ref[...] = jnp.full((_R, _CH), _NEGINF, jnp.float32)
        s_ref[...] = jnp.zeros((_R, _CH), jnp.float32)
        lgw_ref[...] = jnp.zeros((_R, _CH), jnp.float32)

    lane = lax.broadcasted_iota(jnp.int32, (_R, _CH), 1)
    gcol = lane + j * _CH
    rowv = lax.broadcasted_iota(jnp.int32, (_R, _CH), 0) + i * _R
    flat = (rowv * _V + gcol).astype(jnp.uint32)

    lg_c = lg_ref[...]
    bits = _gumbel_bits(flat)
    fl = lax.bitcast_convert_type(
        (bits >> np.uint32(9)) | np.uint32(0x3F800000), jnp.float32) - 1.0
    t = lg_c - jnp.log(-jnp.log(fl + _TINY))

    valid = gcol < _V
    t = jnp.where(valid, t, _NEGINF)
    lg_e = jnp.where(valid, lg_c, _NEGINF)

    tv = tv_ref[...]
    upd = t > tv
    col_ref[...] = jnp.where(upd, gcol, col_ref[...])
    lgw_ref[...] = jnp.where(upd, lg_c, lgw_ref[...])
    tv_ref[...] = jnp.where(upd, t, tv)

    lm = lm_ref[...]
    m_new = jnp.maximum(lm, lg_e)
    s_ref[...] = s_ref[...] * jnp.exp(lm - m_new) + jnp.exp(lg_e - m_new)
    lm_ref[...] = m_new

    @pl.when(j == _NV - 1)
    def _finish():
        tv = tv_ref[...]
        colv = col_ref[...]
        lm = lm_ref[...]
        M = jnp.max(tv, axis=-1, keepdims=True)
        samp = jnp.min(jnp.where(tv == M, colv, _V), axis=-1, keepdims=True)
        m_l = jnp.max(lm, axis=-1, keepdims=True)
        s = jnp.sum(s_ref[...] * jnp.exp(lm - m_l), axis=-1, keepdims=True)
        chosen = jnp.sum(
            jnp.where((tv == M) & (colv == samp), lgw_ref[...], 0.0),
            axis=-1, keepdims=True)
        lp_row = (chosen - m_l) - jnp.log(s)
        samp_ref[0] = samp
        lp_ref[0] = jnp.full((_R, 1), jnp.sum(lp_row), jnp.float32)


def kernel(logits):
    lg = logits.reshape(_NROWS, _V)
    nblk = _NROWS // _R
    samp, lp = pl.pallas_call(
        _body,
        grid=(nblk, _NV),
        in_specs=[pl.BlockSpec((_R, _CH), lambda i, j: (i, j))],
        out_specs=[
            pl.BlockSpec((1, _R, 1), lambda i, j: (i, 0, 0)),
            pl.BlockSpec((1, _R, 1), lambda i, j: (i, 0, 0)),
        ],
        out_shape=[
            jax.ShapeDtypeStruct((nblk, _R, 1), jnp.int32),
            jax.ShapeDtypeStruct((nblk, _R, 1), jnp.float32),
        ],
        scratch_shapes=[
            pltpu.VMEM((_R, _CH), jnp.float32),
            pltpu.VMEM((_R, _CH), jnp.int32),
            pltpu.VMEM((_R, _CH), jnp.float32),
            pltpu.VMEM((_R, _CH), jnp.float32),
            pltpu.VMEM((_R, _CH), jnp.float32),
        ],
    )(lg)
    sample = samp[..., 0]
    log_prob = lp[:, 0, 0]
    return sample, log_prob


# speedup vs baseline: 2.4846x; 2.4846x over previous
import numpy as np
import jax
import jax.numpy as jnp
from jax import lax
from jax.experimental import pallas as pl
from jax.experimental.pallas import tpu as pltpu

_V = 100000
_R = 8
_NROWS = 512
_W = 512
_N1 = 97
_OFF512 = 99328
_OFF128 = 99840
_OFF32 = 99968
_N2 = 195

_ROT = ((13, 15, 26, 6), (17, 29, 16, 24),
        (13, 15, 26, 6), (17, 29, 16, 24),
        (13, 15, 26, 6))

_KS = (np.uint32(0), np.uint32(42),
       np.uint32(np.uint32(42) ^ np.uint32(0x1BD11BDA)))

_TINY = np.float32(np.finfo(np.float32).tiny)
_NEGINF = np.float32(-np.inf)


def _rotl(x, r):
    return (x << np.uint32(r)) | (x >> np.uint32(32 - r))


def _gumbel_bits(t0):
    x0 = t0
    x1 = _rotl(t0, 13) ^ t0
    first = True
    for g in range(5):
        for r in _ROT[g][1 if first else 0:]:
            x0 = x0 + x1
            x1 = _rotl(x1, r) ^ x0
        first = False
        x0 = x0 + _KS[(g + 1) % 3]
        x1 = x1 + _KS[(g + 2) % 3] + np.uint32(g + 1)
    return x0 ^ x1


def _gumbel_shifted(lg_c, t0):
    bits = _gumbel_bits(t0)
    fl = lax.bitcast_convert_type(
        (bits >> np.uint32(9)) | np.uint32(0x3F800000), jnp.float32) - 1.0
    return lg_c - jnp.log(-jnp.log(fl + _TINY))


def _iotas(w, rowbase):
    lane = lax.broadcasted_iota(jnp.int32, (_R, w), 1)
    rowv = lax.broadcasted_iota(jnp.int32, (_R, w), 0) + rowbase
    return lane, (rowv * _V + lane).astype(jnp.uint32)


def _body(lg_ref, samp_ref, lp_ref, tv_ref, col_ref, lm_ref, lgw_ref):
    i = pl.program_id(0)
    rowbase = i * _R

    lane_w, flat_w = _iotas(_W, rowbase)
    flat_w = flat_w + _KS[1]

    tv_ref[...] = jnp.full((_R, 2 * _W), _NEGINF, jnp.float32)
    col_ref[...] = jnp.zeros((_R, 2 * _W), jnp.int32)
    lm_ref[...] = jnp.full((_R, 2 * _W), _NEGINF, jnp.float32)
    lgw_ref[...] = jnp.zeros((_R, 2 * _W), jnp.float32)

    def chain_update(off, sl):
        lg_c = lg_ref[:, pl.ds(off, _W)]
        t = _gumbel_shifted(lg_c, flat_w + jnp.uint32(off))
        gcol = lane_w + off
        tv = tv_ref[:, sl]
        upd = t > tv
        col_ref[:, sl] = jnp.where(upd, gcol, col_ref[:, sl])
        lgw_ref[:, sl] = jnp.where(upd, lg_c, lgw_ref[:, sl])
        tv_ref[:, sl] = jnp.where(upd, t, tv)
        lm_ref[:, sl] = jnp.maximum(lm_ref[:, sl], lg_c)

    sl0 = slice(0, _W)
    sl1 = slice(_W, 2 * _W)

    def loop1(c, carry):
        off0 = c * (2 * _W)
        chain_update(off0, sl0)
        chain_update(off0 + _W, sl1)
        return carry

    lax.fori_loop(0, _N1, loop1, 0)
    chain_update(_OFF512, sl0)

    def piece(off, w):
        lane, flat = _iotas(w, rowbase)
        lg_c = lg_ref[:, pl.ds(off, w)]
        t = _gumbel_shifted(lg_c, flat + (_KS[1] + np.uint32(off)))
        return t, lg_c, lane + off

    t_rem, lg_rem, col_rem = piece(_OFF128, 128)
    t_tail, lg_tail, col_tail = piece(_OFF32, 32)

    red_max = lambda x: jnp.max(x, axis=-1, keepdims=True)
    red_sum = lambda x: jnp.sum(x, axis=-1, keepdims=True)

    tv = tv_ref[...]
    colv = col_ref[...]
    M = jnp.maximum(jnp.maximum(red_max(tv), red_max(t_rem)),
                    red_max(t_tail))
    cand = lambda t, c: jnp.min(jnp.where(t == M, c, _V), axis=-1,
                                keepdims=True)
    samp = jnp.minimum(jnp.minimum(cand(tv, colv), cand(t_rem, col_rem)),
                       cand(t_tail, col_tail))

    m_l = jnp.maximum(jnp.maximum(red_max(lm_ref[...]), red_max(lg_rem)),
                      red_max(lg_tail))

    def loop2(c, acc_s):
        lg_c = lg_ref[:, pl.ds(c * _W, _W)]
        return acc_s + jnp.exp(lg_c - m_l)

    acc_s = lax.fori_loop(0, _N2, loop2, jnp.zeros((_R, _W), jnp.float32))
    s = (red_sum(acc_s) + red_sum(jnp.exp(lg_rem - m_l))
         + red_sum(jnp.exp(lg_tail - m_l)))

    chosen = (red_sum(jnp.where((tv == M) & (colv == samp), lgw_ref[...], 0.0))
              + red_sum(jnp.where(col_rem == samp, lg_rem, 0.0))
              + red_sum(jnp.where(col_tail == samp, lg_tail, 0.0)))
    lp_row = (chosen - m_l) - jnp.log(s)

    samp_ref[0] = samp
    lp_ref[0] = jnp.full((_R, 1), jnp.sum(lp_row), jnp.float32)


def kernel(logits):
    lg = logits.reshape(_NROWS, _V)
    nblk = _NROWS // _R
    samp, lp = pl.pallas_call(
        _body,
        grid=(nblk,),
        in_specs=[pl.BlockSpec((_R, _V), lambda i: (i, 0))],
        out_specs=[
            pl.BlockSpec((1, _R, 1), lambda i: (i, 0, 0)),
            pl.BlockSpec((1, _R, 1), lambda i: (i, 0, 0)),
        ],
        out_shape=[
            jax.ShapeDtypeStruct((nblk, _R, 1), jnp.int32),
            jax.ShapeDtypeStruct((nblk, _R, 1), jnp.float32),
        ],
        scratch_shapes=[
            pltpu.VMEM((_R, 2 * _W), jnp.float32),
            pltpu.VMEM((_R, 2 * _W), jnp.int32),
            pltpu.VMEM((_R, 2 * _W), jnp.float32),
            pltpu.VMEM((_R, 2 * _W), jnp.float32),
        ],
    )(lg)
    sample = samp[..., 0]
    log_prob = lp[:, 0, 0]
    return sample, log_prob


# speedup vs baseline: 5.2435x; 2.1104x over previous
import numpy as np
import jax
import jax.numpy as jnp
from jax import lax
from jax.experimental import pallas as pl
from jax.experimental.pallas import tpu as pltpu

_V = 100000
_R = 8
_RB = 16
_NROWS = 512
_W = 512
_NU = 12
_OFFA = 98304
_OFFB = 98816
_OFF512 = 99328
_OFF128 = 99840
_OFF32 = 99968

_ROT = ((13, 15, 26, 6), (17, 29, 16, 24),
        (13, 15, 26, 6), (17, 29, 16, 24),
        (13, 15, 26, 6))

_KS = (np.uint32(0), np.uint32(42),
       np.uint32(np.uint32(42) ^ np.uint32(0x1BD11BDA)))

_TINY = np.float32(np.finfo(np.float32).tiny)
_NEGINF = np.float32(-np.inf)


def _rotl(x, r):
    return (x << np.uint32(r)) | (x >> np.uint32(32 - r))


def _gumbel_bits(t0):
    x0 = t0
    x1 = _rotl(t0, 13) ^ t0
    first = True
    for g in range(5):
        for r in _ROT[g][1 if first else 0:]:
            x0 = x0 + x1
            x1 = _rotl(x1, r) ^ x0
        first = False
        x0 = x0 + _KS[(g + 1) % 3]
        x1 = x1 + _KS[(g + 2) % 3] + np.uint32(g + 1)
    return x0 ^ x1


def _gumbel_shifted(lg_c, t0):
    bits = _gumbel_bits(t0)
    fl = lax.bitcast_convert_type(
        (bits >> np.uint32(9)) | np.uint32(0x3F800000), jnp.float32) - 1.0
    return lg_c - jnp.log(-jnp.log(fl + _TINY))


def _iotas(w, rowbase):
    lane = lax.broadcasted_iota(jnp.int32, (_R, w), 1)
    rowv = lax.broadcasted_iota(jnp.int32, (_R, w), 0) + rowbase
    return lane, (rowv * _V + lane).astype(jnp.uint32)


def _body(lg_ref, samp_ref, lp_ref, tv_ref, col_ref, lm_ref, lgw_ref, s_ref):
    i = pl.program_id(0)
    rowbase = i * _RB

    lane_w, flat_w = _iotas(_W, rowbase)
    flat_w = flat_w + _KS[1]

    tv_ref[...] = jnp.full((_RB, 2 * _W), _NEGINF, jnp.float32)
    col_ref[...] = jnp.zeros((_RB, 2 * _W), jnp.int32)
    lm_ref[...] = jnp.full((_RB, 2 * _W), _NEGINF, jnp.float32)
    lgw_ref[...] = jnp.zeros((_RB, 2 * _W), jnp.float32)
    s_ref[...] = jnp.zeros((_RB, 2 * _W), jnp.float32)

    rs0 = slice(0, _R)
    rs1 = slice(_R, 2 * _R)
    sl0 = slice(0, _W)
    sl1 = slice(_W, 2 * _W)

    def update(t, lg_c, gcol, rs, sl):
        tv = tv_ref[rs, sl]
        upd = t > tv
        col_ref[rs, sl] = jnp.where(upd, gcol, col_ref[rs, sl])
        lgw_ref[rs, sl] = jnp.where(upd, lg_c, lgw_ref[rs, sl])
        tv_ref[rs, sl] = jnp.where(upd, t, tv)
        lm = lm_ref[rs, sl]
        m_new = jnp.maximum(lm, lg_c)
        s_ref[rs, sl] = (s_ref[rs, sl] * jnp.exp(lm - m_new)
                         + jnp.exp(lg_c - m_new))
        lm_ref[rs, sl] = m_new

    def chain_update(off, h, sl):
        rs = rs0 if h == 0 else rs1
        lg_c = lg_ref[rs, pl.ds(off, _W)]
        t = _gumbel_shifted(lg_c, flat_w + jnp.uint32(off + h * _R * _V))
        update(t, lg_c, lane_w + off, rs, sl)

    def loop1(c, carry):
        off0 = c * (8 * _W)
        for h in range(2):
            for k in range(8):
                chain_update(off0 + k * _W, h, sl0 if k % 2 == 0 else sl1)
        return carry

    lax.fori_loop(0, _NU, loop1, 0)
    for h in range(2):
        chain_update(_OFFA, h, sl0)
        chain_update(_OFFB, h, sl1)
        chain_update(_OFF512, h, sl0)

    def piece(off, w, h, sl):
        rs = rs0 if h == 0 else rs1
        lane, flat = _iotas(w, rowbase + h * _R)
        lg_c = lg_ref[rs, pl.ds(off, w)]
        t = _gumbel_shifted(lg_c, flat + (_KS[1] + np.uint32(off)))
        update(t, lg_c, lane + off, rs, sl)

    for h in range(2):
        piece(_OFF128, 128, h, slice(0, 128))
        piece(_OFF32, 32, h, slice(0, 32))

    def fold(x, op):
        parts = [x[:, k * 128:(k + 1) * 128] for k in range(x.shape[1] // 128)]
        acc = parts[0]
        for p in parts[1:]:
            acc = op(acc, p)
        return acc

    red_max = lambda x: jnp.max(fold(x, jnp.maximum), axis=-1, keepdims=True)
    red_min = lambda x: jnp.min(fold(x, jnp.minimum), axis=-1, keepdims=True)
    red_sum = lambda x: jnp.sum(fold(x, jnp.add), axis=-1, keepdims=True)

    tv = tv_ref[...]
    colv = col_ref[...]
    M = red_max(tv)
    samp = red_min(jnp.where(tv == M, colv, _V))
    lm = lm_ref[...]
    m_l = red_max(lm)
    s = red_sum(s_ref[...] * jnp.exp(lm - m_l))

    chosen = red_sum(
        jnp.where((tv == M) & (colv == samp), lgw_ref[...], 0.0))
    lp_row = (chosen - m_l) - jnp.log(s)

    samp_ref[0] = samp
    lp_a = jnp.sum(lp_row[rs0])
    lp_b = jnp.sum(lp_row[rs1])
    lp_ref[0] = jnp.concatenate(
        [jnp.full((_R, 1), lp_a, jnp.float32),
         jnp.full((_R, 1), lp_b, jnp.float32)], axis=0)


def kernel(logits):
    lg = logits.reshape(_NROWS, _V)
    nblk = _NROWS // _RB
    samp, lp = pl.pallas_call(
        _body,
        grid=(nblk,),
        in_specs=[pl.BlockSpec((_RB, _V), lambda i: (i, 0))],
        out_specs=[
            pl.BlockSpec((1, _RB, 1), lambda i: (i, 0, 0)),
            pl.BlockSpec((1, _RB, 1), lambda i: (i, 0, 0)),
        ],
        out_shape=[
            jax.ShapeDtypeStruct((nblk, _RB, 1), jnp.int32),
            jax.ShapeDtypeStruct((nblk, _RB, 1), jnp.float32),
        ],
        scratch_shapes=[
            pltpu.VMEM((_RB, 2 * _W), jnp.float32),
            pltpu.VMEM((_RB, 2 * _W), jnp.int32),
            pltpu.VMEM((_RB, 2 * _W), jnp.float32),
            pltpu.VMEM((_RB, 2 * _W), jnp.float32),
            pltpu.VMEM((_RB, 2 * _W), jnp.float32),
        ],
    )(lg)
    sample = samp[..., 0].reshape(64, 8)
    log_prob = lp[:, ::_R, 0].reshape(64)
    return sample, log_prob
